# Initial kernel scaffold; baseline (speedup 1.0000x reference)
import jax
import jax.numpy as jnp
from jax.experimental import pallas as pl


def kernel(preds, targets):
    raise NotImplementedError("write your pallas kernel here")


# speedup vs baseline: 17.5229x; 17.5229x over previous
import functools

import numpy as np
import jax
import jax.numpy as jnp
from jax import lax
from jax.experimental import pallas as pl
from jax.experimental.pallas import tpu as pltpu
from jax.experimental.pallas import tpu_sc as plsc

N = 65536
D = 64
K = int(N * 0.1)

NUM_CORES = 2
NUM_SUBCORES = 16
LANES = 16
NWORKERS = NUM_CORES * NUM_SUBCORES
COLS_PER_W = D // NWORKERS

NVEC = N // LANES
UNROLL = 4

MIN32 = np.int32(-(2**31))


def _sc_select_body(bits_hbm, out_hbm, colbuf, hist, merged, rowbuf):
  wid = lax.axis_index("s") * NUM_CORES + lax.axis_index("c")
  lane = lax.iota(jnp.int32, LANES)
  ones = jnp.ones((LANES,), jnp.int32)
  zeros16 = jnp.zeros((LANES,), jnp.int32)

  def clear_hist(_i, carry):
    for l in range(LANES):
      hist[l, pl.ds(_i * LANES, LANES)] = zeros16
    return carry

  def merge_and_clear(nbins):
    def body(j, carry):
      acc = zeros16
      for l in range(LANES):
        acc = acc + hist[l, pl.ds(j * LANES, LANES)]
        hist[l, pl.ds(j * LANES, LANES)] = zeros16
      merged[pl.ds(j * LANES, LANES)] = acc
      return carry
    lax.fori_loop(0, nbins // LANES, body, 0, unroll=2)

  def extract(vec, idx):
    return jnp.sum(jnp.where(lane == idx, vec, 0))

  def walk_desc(off, nbins, r):
    def body(jj, carry):
      acc, beta, rnew, done = carry
      j = nbins // LANES - 1 - jj
      vec = merged[pl.ds(off + j * LANES, LANES)]
      rv = lax.rev(vec, (0,))
      cs = jnp.cumsum(rv)
      cond = (acc + cs) >= r
      lam = jnp.sum(jnp.where(cond, 0, 1))
      hit = jnp.logical_and(lam < LANES, done == 0)
      csl = extract(cs, lam)
      rvl = extract(rv, lam)
      beta = jnp.where(hit, j * LANES + (LANES - 1) - lam, beta)
      rnew = jnp.where(hit, r - (acc + csl - rvl), rnew)
      done = jnp.where(hit, 1, done)
      acc = acc + jnp.sum(vec)
      return acc, beta, rnew, done
    z = jnp.int32(0)
    _, beta, rnew, _ = lax.fori_loop(0, nbins // LANES, body, (z, z, z, z))
    return beta, rnew

  def walk_asc(off, nbins, r):
    def body(j, carry):
      acc, beta, rnew, done = carry
      vec = merged[pl.ds(off + j * LANES, LANES)]
      cs = jnp.cumsum(vec)
      cond = (acc + cs) >= r
      lam = jnp.sum(jnp.where(cond, 0, 1))
      hit = jnp.logical_and(lam < LANES, done == 0)
      csl = extract(cs, lam)
      vl = extract(vec, lam)
      beta = jnp.where(hit, j * LANES + lam, beta)
      rnew = jnp.where(hit, r - (acc + csl - vl), rnew)
      done = jnp.where(hit, 1, done)
      acc = acc + jnp.sum(vec)
      return acc, beta, rnew, done
    z = jnp.int32(0)
    _, beta, rnew, _ = lax.fori_loop(0, nbins // LANES, body, (z, z, z, z))
    return beta, rnew

  lax.fori_loop(0, 2048 // LANES, clear_hist, 0)

  def per_column(j, carry):
    col = wid * COLS_PER_W + j
    pltpu.sync_copy(bits_hbm.at[col], colbuf)

    def scan1(i, carry):
      for t in range(UNROLL):
        sl = pl.ds((i * UNROLL + t) * LANES, LANES)
        b = colbuf[sl]
        s = lax.shift_right_arithmetic(b, 31)
        u = b ^ (s | MIN32)
        colbuf[sl] = u
        b1 = lax.shift_right_logical(u, 21)
        plsc.addupdate_scatter(hist, [lane, b1], ones)
      return carry
    lax.fori_loop(0, NVEC // UNROLL, scan1, 0)
    merge_and_clear(2048)
    p_t, r_t = walk_desc(0, 2048, jnp.int32(K))
    p_b, r_b = walk_asc(0, 2048, jnp.int32(K))

    def refine(shift_p, shift_b, p_t, p_b, r_t, r_b):
      def scan(i, carry):
        for t in range(UNROLL):
          sl = pl.ds((i * UNROLL + t) * LANES, LANES)
          u = colbuf[sl]
          pref = lax.shift_right_logical(u, shift_p)
          bb = lax.shift_right_logical(u, shift_b) & 1023
          plsc.addupdate_scatter(hist, [lane, bb], ones, mask=pref == p_t)
          plsc.addupdate_scatter(hist, [lane, bb + 1024], ones,
                                 mask=pref == p_b)
        return carry
      lax.fori_loop(0, NVEC // UNROLL, scan, 0)
      merge_and_clear(2048)
      bt, r_t = walk_desc(0, 1024, r_t)
      bb_, r_b = walk_asc(1024, 1024, r_b)
      return p_t * 1024 + bt, p_b * 1024 + bb_, r_t, r_b

    p_t, p_b, r_t, r_b = refine(21, 11, p_t, p_b, r_t, r_b)
    p_t, p_b, r_t, r_b = refine(11, 1, p_t, p_b, r_t, r_b)

    val_t_hi = lax.shift_left(p_t, 1) | 1
    val_b_lo = lax.shift_left(p_b, 1)

    def scan4(i, carry):
      acc_t, acc_b = carry
      for t in range(UNROLL):
        sl = pl.ds((i * UNROLL + t) * LANES, LANES)
        u = colbuf[sl]
        acc_t = acc_t + jnp.where(u == val_t_hi, 1, 0)
        acc_b = acc_b + jnp.where(u == val_b_lo, 1, 0)
      return acc_t, acc_b
    acc_t, acc_b = lax.fori_loop(0, NVEC // UNROLL, scan4,
                                 (zeros16, zeros16))
    n_hi_t = jnp.sum(acc_t)
    n_lo_b = jnp.sum(acc_b)
    ut = lax.shift_left(p_t, 1) | jnp.where(r_t <= n_hi_t, 1, 0)
    ub = lax.shift_left(p_b, 1) | jnp.where(r_b <= n_lo_b, 0, 1)
    vt = ut ^ MIN32
    vb = ub ^ MIN32
    rowbuf[...] = jnp.where(lane == 0, vt, jnp.where(lane == 1, vb, 0))
    pltpu.sync_copy(rowbuf, out_hbm.at[col])
    return carry

  lax.fori_loop(0, COLS_PER_W, per_column, 0)


def _sc_select(bits_t):
  mesh = plsc.VectorSubcoreMesh(core_axis_name="c", subcore_axis_name="s")
  return pl.kernel(
      _sc_select_body,
      out_type=jax.ShapeDtypeStruct((D, LANES), jnp.int32),
      mesh=mesh,
      compiler_params=pltpu.CompilerParams(needs_layout_passes=False),
      scratch_types=[
          pltpu.VMEM((N,), jnp.int32),
          pltpu.VMEM((LANES, 2048), jnp.int32),
          pltpu.VMEM((2048,), jnp.int32),
          pltpu.VMEM((LANES,), jnp.int32),
      ],
  )(bits_t)


ROWS_PER_BLK = 4096
NBLK = N // ROWS_PER_BLK


def _tc_reduce_body(thr_ref, p_ref, t_ref, o_ref, acc_ref):
  i = pl.program_id(0)

  @pl.when(i == 0)
  def _():
    acc_ref[...] = jnp.zeros((8, D), jnp.float32)

  b = lax.bitcast_convert_type(p_ref[...], jnp.int32)
  v = jnp.where(b >= 0, b, b ^ jnp.int32(0x7FFFFFFF))
  tgt = t_ref[...]
  t_t = thr_ref[0:1, :]
  t_b = thr_ref[1:2, :]
  gt = v > t_t
  eqt = v == t_t
  ltb = v < t_b
  eqb = v == t_b

  def srow(m, x):
    return jnp.sum(jnp.where(m, x, 0.0), axis=0, keepdims=True)

  def crow(m):
    return jnp.sum(m.astype(jnp.float32), axis=0, keepdims=True)

  upd = jnp.concatenate([
      srow(gt, tgt), crow(gt), srow(eqt, tgt), crow(eqt),
      srow(ltb, tgt), crow(ltb), srow(eqb, tgt), crow(eqb),
  ], axis=0)
  acc_ref[...] += upd

  @pl.when(i == NBLK - 1)
  def _():
    a = acc_ref[...]
    kf = jnp.float32(K)
    top = a[0:1] + (kf - a[1:2]) * a[2:3] / a[3:4]
    bot = a[4:5] + (kf - a[5:6]) * a[6:7] / a[7:8]
    ret = (top - bot) / kf
    o_ref[...] = jnp.concatenate(
        [ret, jnp.zeros((7, D), jnp.float32)], axis=0)


def _tc_reduce(thr8, preds, targets):
  return pl.pallas_call(
      _tc_reduce_body,
      grid=(NBLK,),
      in_specs=[
          pl.BlockSpec((8, D), lambda i: (0, 0)),
          pl.BlockSpec((ROWS_PER_BLK, D), lambda i: (i, 0)),
          pl.BlockSpec((ROWS_PER_BLK, D), lambda i: (i, 0)),
      ],
      out_specs=pl.BlockSpec((8, D), lambda i: (0, 0)),
      out_shape=jax.ShapeDtypeStruct((8, D), jnp.float32),
      scratch_shapes=[pltpu.VMEM((8, D), jnp.float32)],
  )(thr8, preds, targets)


@jax.jit
def kernel(preds, targets):
  bits_t = lax.bitcast_convert_type(preds, jnp.int32).T
  thr = _sc_select(bits_t)
  thr8 = jnp.zeros((8, D), jnp.int32).at[0].set(thr[:, 0]).at[1].set(thr[:, 1])
  out = _tc_reduce(thr8, preds, targets)
  return out[0]


# speedup vs baseline: 50.5622x; 2.8855x over previous
import numpy as np
import jax
import jax.numpy as jnp
from jax import lax
from jax.experimental import pallas as pl
from jax.experimental.pallas import tpu as pltpu
from jax.experimental.pallas import tpu_sc as plsc

N = 65536
D = 64
K = int(N * 0.1)

NUM_CORES = 2
NUM_SUBCORES = 16
LANES = 16
NWORKERS = NUM_CORES * NUM_SUBCORES
COLS_PER_W = D // NWORKERS

NVEC = N // LANES
MIN32 = np.int32(-(2**31))


def _sc_select_body(bits_hbm, out_hbm, colbuf, hist, persum, rowbuf):
  wid = lax.axis_index("s") * NUM_CORES + lax.axis_index("c")
  lane = lax.iota(jnp.int32, LANES)
  ones = jnp.ones((LANES,), jnp.int32)
  zeros16 = jnp.zeros((LANES,), jnp.int32)

  def extract(vec, idx):
    return jnp.sum(jnp.where(lane == idx, vec, 0))

  def desc_scan(get, n16, r):
    def body(jj, c):
      acc, idx, rn, cnt, done = c
      j = n16 - 1 - jj
      vec = get(j)
      rv = lax.rev(vec, (0,))
      cs = jnp.cumsum(rv)
      cond = (acc + cs) >= r
      lam = jnp.sum(jnp.where(cond, 0, 1))
      hit = jnp.logical_and(lam < LANES, done == 0)
      csl = extract(cs, lam)
      rvl = extract(rv, lam)
      idx = jnp.where(hit, j * LANES + (LANES - 1) - lam, idx)
      rn = jnp.where(hit, r - (acc + csl - rvl), rn)
      cnt = jnp.where(hit, rvl, cnt)
      done = jnp.where(hit, 1, done)
      return acc + jnp.sum(vec), idx, rn, cnt, done
    z = jnp.int32(0)
    _, idx, rn, cnt, _ = lax.fori_loop(0, n16, body, (z, z, z, z, z))
    return idx, rn, cnt

  def asc_scan(get, n16, r):
    def body(j, c):
      acc, idx, rn, cnt, done = c
      vec = get(j)
      cs = jnp.cumsum(vec)
      cond = (acc + cs) >= r
      lam = jnp.sum(jnp.where(cond, 0, 1))
      hit = jnp.logical_and(lam < LANES, done == 0)
      csl = extract(cs, lam)
      vl = extract(vec, lam)
      idx = jnp.where(hit, j * LANES + lam, idx)
      rn = jnp.where(hit, r - (acc + csl - vl), rn)
      cnt = jnp.where(hit, vl, cnt)
      done = jnp.where(hit, 1, done)
      return acc + jnp.sum(vec), idx, rn, cnt, done
    z = jnp.int32(0)
    _, idx, rn, cnt, _ = lax.fori_loop(0, n16, body, (z, z, z, z, z))
    return idx, rn, cnt

  def build_persum(nbins):
    @plsc.parallel_loop(0, nbins // LANES, unroll=8)
    def _(j):
      s = jnp.sum(hist[pl.ds(j * LANES, LANES)])
      plsc.store_scatter(persum, [jnp.full((LANES,), j, jnp.int32)],
                         jnp.full((LANES,), s, jnp.int32), mask=lane == 0)

  def clear_hist(nwords):
    @plsc.parallel_loop(0, nwords // LANES, unroll=8)
    def _(j):
      hist[pl.ds(j * LANES, LANES)] = zeros16

  def walk_desc(off, nbins, psoff, r):
    nch = nbins // LANES
    c, rc, _ = desc_scan(
        lambda j: persum[pl.ds(psoff + j * LANES, LANES)], nch // LANES, r)
    b, rn, cnt = desc_scan(
        lambda j: hist[pl.ds(off + c * LANES, LANES)], 1, rc)
    return c * LANES + b, rn, cnt

  def walk_asc(off, nbins, psoff, r):
    nch = nbins // LANES
    c, rc, _ = asc_scan(
        lambda j: persum[pl.ds(psoff + j * LANES, LANES)], nch // LANES, r)
    b, rn, cnt = asc_scan(
        lambda j: hist[pl.ds(off + c * LANES, LANES)], 1, rc)
    return c * LANES + b, rn, cnt

  def per_column(jcol, carry):
    col = wid * COLS_PER_W + jcol
    pltpu.sync_copy(bits_hbm.at[col], colbuf)
    clear_hist(4096)

    @plsc.parallel_loop(0, NVEC, unroll=8)
    def _(i):
      sl = pl.ds(i * LANES, LANES)
      b = colbuf[sl]
      s = lax.shift_right_arithmetic(b, 31)
      u = b ^ (s | MIN32)
      colbuf[sl] = u
      plsc.addupdate_scatter(hist, [lax.shift_right_logical(u, 21)], ones)

    build_persum(2048)
    p1t, r1t, _ = walk_desc(0, 2048, 0, jnp.int32(K))
    p1b, r1b, _ = walk_asc(0, 2048, 0, jnp.int32(K))
    clear_hist(4096)

    @plsc.parallel_loop(0, NVEC, unroll=8)
    def _(i):
      u = colbuf[pl.ds(i * LANES, LANES)]
      pref = lax.shift_right_logical(u, 21)
      bb = lax.shift_right_logical(u, 10) & 2047
      plsc.addupdate_scatter(hist, [bb], ones, mask=pref == p1t)
      plsc.addupdate_scatter(hist, [bb + 2048], ones, mask=pref == p1b)

    build_persum(4096)
    bt, r2t, _ = walk_desc(0, 2048, 0, r1t)
    bb_, r2b, _ = walk_asc(2048, 2048, 128, r1b)
    p2t = p1t * 2048 + bt
    p2b = p1b * 2048 + bb_
    clear_hist(2048)

    @plsc.parallel_loop(0, NVEC, unroll=8)
    def _(i):
      u = colbuf[pl.ds(i * LANES, LANES)]
      pref = lax.shift_right_logical(u, 10)
      bb = u & 1023
      plsc.addupdate_scatter(hist, [bb], ones, mask=pref == p2t)
      plsc.addupdate_scatter(hist, [bb + 1024], ones, mask=pref == p2b)

    build_persum(2048)
    bt, r3t, n_t = walk_desc(0, 1024, 0, r2t)
    bb_, r3b, n_b = walk_asc(1024, 1024, 64, r2b)
    ut = lax.shift_left(p2t, 10) | bt
    ub = lax.shift_left(p2b, 10) | bb_

    a_t = jnp.full((LANES,), r3t, jnp.int32).astype(jnp.float32) / \
        jnp.full((LANES,), n_t, jnp.int32).astype(jnp.float32)
    a_b = jnp.full((LANES,), r3b, jnp.int32).astype(jnp.float32) / \
        jnp.full((LANES,), n_b, jnp.int32).astype(jnp.float32)

    row = jnp.where(lane == 0, ut ^ MIN32, jnp.where(lane == 1, ub ^ MIN32, 0))
    row = jnp.where(lane == 2, plsc.bitcast(a_t, jnp.int32), row)
    row = jnp.where(lane == 3, plsc.bitcast(a_b, jnp.int32), row)
    rowbuf[...] = row
    pltpu.sync_copy(rowbuf, out_hbm.at[col])
    return carry

  lax.fori_loop(0, COLS_PER_W, per_column, 0)


def _sc_select(bits_t):
  mesh = plsc.VectorSubcoreMesh(core_axis_name="c", subcore_axis_name="s")
  return pl.kernel(
      _sc_select_body,
      out_type=jax.ShapeDtypeStruct((D, LANES), jnp.int32),
      mesh=mesh,
      compiler_params=pltpu.CompilerParams(needs_layout_passes=False),
      scratch_types=[
          pltpu.VMEM((N,), jnp.int32),
          pltpu.VMEM((4096,), jnp.int32),
          pltpu.VMEM((256,), jnp.int32),
          pltpu.VMEM((LANES,), jnp.int32),
      ],
  )(bits_t)


ROWS_PER_BLK = 4096
NBLK = N // ROWS_PER_BLK


def _tc_reduce_body(thr_ref, p_ref, t_ref, o_ref, acc_ref):
  i = pl.program_id(0)

  @pl.when(i == 0)
  def _():
    acc_ref[...] = jnp.zeros((8, D), jnp.float32)

  b = lax.bitcast_convert_type(p_ref[...], jnp.int32)
  m = lax.shift_right_logical(lax.shift_right_arithmetic(b, 31), 1)
  v = b ^ m
  tgt = t_ref[...]
  t_t = thr_ref[0:1, :]
  t_b = thr_ref[1:2, :]
  a_t = lax.bitcast_convert_type(thr_ref[2:3, :], jnp.float32)
  a_b = lax.bitcast_convert_type(thr_ref[3:4, :], jnp.float32)
  w = jnp.where(v > t_t, 1.0, jnp.where(v == t_t, a_t, 0.0)) - \
      jnp.where(v < t_b, 1.0, jnp.where(v == t_b, a_b, 0.0))
  part = jnp.sum(w * tgt, axis=0, keepdims=True)
  acc_ref[0:1, :] += part

  @pl.when(i == NBLK - 1)
  def _():
    ret = acc_ref[0:1, :] * jnp.float32(1.0 / K)
    o_ref[...] = jnp.concatenate(
        [ret, jnp.zeros((7, D), jnp.float32)], axis=0)


def _tc_reduce(thr8, preds, targets):
  return pl.pallas_call(
      _tc_reduce_body,
      grid=(NBLK,),
      in_specs=[
          pl.BlockSpec((8, D), lambda i: (0, 0)),
          pl.BlockSpec((ROWS_PER_BLK, D), lambda i: (i, 0)),
          pl.BlockSpec((ROWS_PER_BLK, D), lambda i: (i, 0)),
      ],
      out_specs=pl.BlockSpec((8, D), lambda i: (0, 0)),
      out_shape=jax.ShapeDtypeStruct((8, D), jnp.float32),
      scratch_shapes=[pltpu.VMEM((8, D), jnp.float32)],
  )(thr8, preds, targets)


@jax.jit
def kernel(preds, targets):
  bits_t = lax.bitcast_convert_type(preds, jnp.int32).T
  thr = _sc_select(bits_t)
  thr8 = jnp.concatenate(
      [thr[:, :4].T, jnp.zeros((4, D), jnp.int32)], axis=0)
  out = _tc_reduce(thr8, preds, targets)
  return out[0]
